# baseline (speedup 1.0000x reference)
import functools

import jax
import jax.numpy as jnp
import numpy as np
from jax import lax
from jax.experimental import pallas as pl
from jax.experimental.pallas import tpu as pltpu
from jax.experimental.pallas import tpu_sc as plsc

N = 10000
E = 160000
H = 128
RBF = 16
CUTOFF = 5.0

NP = 10240
PKW = 512
E2 = 162000
EP = 164864
SC_CH = 1792
NCH = EP // SC_CH
RN = 160
C = 32
CAP = 64


def _dense_body(s_ref, W1_ref, b1_ref, W2_ref, b2_ref, out_ref):
    h = s_ref[...] @ W1_ref[...] + b1_ref[...]
    h = h * jax.nn.sigmoid(h)
    out_ref[...] = h @ W2_ref[...] + b2_ref[...]


def _dense(s, W1, b1, W2, b2):
    BN = 1024
    return pl.pallas_call(
        _dense_body,
        grid=(NP // BN,),
        in_specs=[
            pl.BlockSpec((BN, H), lambda i: (i, 0)),
            pl.BlockSpec((H, H), lambda i: (0, 0)),
            pl.BlockSpec((1, H), lambda i: (0, 0)),
            pl.BlockSpec((H, 3 * H), lambda i: (0, 0)),
            pl.BlockSpec((1, 3 * H), lambda i: (0, 0)),
        ],
        out_specs=pl.BlockSpec((BN, 3 * H), lambda i: (i, 0)),
        out_shape=jax.ShapeDtypeStruct((NP, 3 * H), jnp.float32),
    )(s, W1, b1.reshape(1, H), W2, b2.reshape(1, 3 * H))


def _pack_body(phi_ref, d_ref, dir_ref, Wr_ref, br_ref, out_ref):
    r = phi_ref[...] @ Wr_ref[...] + br_ref[...]
    d = d_ref[...]
    cut = 0.5 * (jnp.cos(jnp.pi * d / CUTOFF) + 1.0) * (d < CUTOFF).astype(d.dtype)
    out_ref[:, : 3 * H] = r * cut
    be = r.shape[0]
    out_ref[:, 3 * H: 4 * H] = jnp.zeros((be, H), jnp.float32)
    for k in range(3):
        col = dir_ref[:, k:k + 1]
        out_ref[:, 3 * H + 16 * k: 3 * H + 16 * (k + 1)] = jnp.broadcast_to(
            col, (be, 16))


def _pack(phi, d, dir_ij, Wr, br):
    BE = 2000
    return pl.pallas_call(
        _pack_body,
        grid=(E2 // BE,),
        in_specs=[
            pl.BlockSpec((BE, RBF), lambda i: (i, 0)),
            pl.BlockSpec((BE, 1), lambda i: (i, 0)),
            pl.BlockSpec((BE, 3), lambda i: (i, 0)),
            pl.BlockSpec((RBF, 3 * H), lambda i: (0, 0)),
            pl.BlockSpec((1, 3 * H), lambda i: (0, 0)),
        ],
        out_specs=pl.BlockSpec((BE, PKW), lambda i: (i, 0)),
        out_shape=jax.ShapeDtypeStruct((E2, PKW), jnp.float32),
    )(phi, d, dir_ij, Wr, br.reshape(1, 3 * H))


_DN = lax.GatherDimensionNumbers(
    offset_dims=(), collapsed_slice_dims=(0,), start_index_map=(0,))


def _vperm(x, idx):
    return lax.gather(x, idx[:, None], _DN, (1,),
                      mode=lax.GatherScatterMode.PROMISE_IN_BOUNDS)


def _sc_body(se_hbm, pk_hbm, vp_hbm, sp_hbm, ip_hbm, jp_hbm,
             outv_hbm, outs_hbm,
             ib0, ib1, jb0, jb1, eid_q, dst_q, jq, tmp, ge, gj,
             se_b, pk_b, v_b, acc_dv, acc_ds,
             semi0, semi1, semj0, semj1, sg1, sg2, sg3):
    c = lax.axis_index("c")
    s = lax.axis_index("s")
    w = c * 16 + s
    iot = lax.iota(jnp.int32, 16)

    ibufs = (ib0, ib1)
    jbufs = (jb0, jb1)
    semis = (semi0, semi1)
    semjs = (semj0, semj1)

    def fire(nv):
        for gi in range(C // 16):
            lanes = gi * 16 + iot
            val = lanes < nv
            e16 = jnp.where(val, eid_q[pl.ds(gi * 16, 16)], E)
            j16 = jnp.clip(jq[pl.ds(gi * 16, 16)], 0, NP - 1)
            d16 = jnp.clip(dst_q[pl.ds(gi * 16, 16)], 0, RN - 1)
            ge[pl.ds(gi * 16, 16)] = e16
            gj[pl.ds(gi * 16, 16)] = j16
            dst_q[pl.ds(gi * 16, 16)] = d16
        cp1 = pltpu.async_copy(se_hbm.at[gj], se_b, sg1)
        cp2 = pltpu.async_copy(pk_hbm.at[ge], pk_b, sg2)
        cp3 = pltpu.async_copy(vp_hbm.at[gj], v_b, sg3)
        cp1.wait()
        cp2.wait()
        cp3.wait()

        def sub(gi, _):
            d16 = dst_q[pl.ds(gi * 16, 16)]
            for kk in range(16):
                e = gi * 16 + kk
                dr = d16[kk]
                dirx = pk_b[e, pl.ds(3 * H, 16)]
                diry = pk_b[e, pl.ds(3 * H + 16, 16)]
                dirz = pk_b[e, pl.ds(3 * H + 32, 16)]

                def hcb(hc, _2, e=e, dr=dr, dirx=dirx, diry=diry, dirz=dirz):
                    o = hc * 16
                    g_dvv = se_b[e, pl.ds(o, 16)] * pk_b[e, pl.ds(o, 16)]
                    g_ds = se_b[e, pl.ds(H + o, 16)] * pk_b[e, pl.ds(H + o, 16)]
                    g_dvr = se_b[e, pl.ds(2 * H + o, 16)] * pk_b[e, pl.ds(2 * H + o, 16)]
                    plsc.addupdate(acc_ds.at[dr, pl.ds(o, 16)], g_ds)
                    plsc.addupdate(acc_dv.at[dr, pl.ds(o, 16)],
                                   g_dvr * dirx + g_dvv * v_b[e, pl.ds(o, 16)])
                    plsc.addupdate(acc_dv.at[dr, pl.ds(H + o, 16)],
                                   g_dvr * diry + g_dvv * v_b[e, pl.ds(H + o, 16)])
                    plsc.addupdate(acc_dv.at[dr, pl.ds(2 * H + o, 16)],
                                   g_dvr * dirz + g_dvv * v_b[e, pl.ds(2 * H + o, 16)])
                    return 0

                lax.fori_loop(0, H // 16, hcb, 0)
            return 0

        lax.fori_loop(0, C // 16, sub, 0)

    def one_pass(p, _):
        lo = (p * 32 + w) * RN
        pltpu.sync_copy(vp_hbm.at[pl.ds(lo, RN)], acc_dv)
        pltpu.sync_copy(sp_hbm.at[pl.ds(lo, RN)], acc_ds)
        pltpu.async_copy(ip_hbm.at[pl.ds(0, SC_CH)], ib0, semi0)
        pltpu.async_copy(jp_hbm.at[pl.ds(0, SC_CH)], jb0, semj0)

        def step(st, fill0):
            fill = fill0
            for par in range(2):
                k = st * 2 + par
                ibuf = ibufs[par]
                jbuf = jbufs[par]
                pltpu.make_async_copy(
                    ip_hbm.at[pl.ds(k * SC_CH, SC_CH)], ibuf, semis[par]).wait()
                pltpu.make_async_copy(
                    jp_hbm.at[pl.ds(k * SC_CH, SC_CH)], jbuf, semjs[par]).wait()

                @pl.when(k + 1 < NCH)
                def _():
                    nxt = (k + 1) * SC_CH
                    pltpu.async_copy(
                        ip_hbm.at[pl.ds(nxt, SC_CH)], ibufs[1 - par], semis[1 - par])
                    pltpu.async_copy(
                        jp_hbm.at[pl.ds(nxt, SC_CH)], jbufs[1 - par], semjs[1 - par])

                def grp(g, fll):
                    iv = ibuf[pl.ds(g * 16, 16)]
                    dstv = iv - lo
                    mask = (dstv >= 0) & (dstv < RN)
                    cs = jnp.where(mask, 1, 0)
                    for sh in (1, 2, 4, 8):
                        prev = _vperm(cs, jnp.maximum(iot - sh, 0))
                        cs = cs + jnp.where(iot >= sh, prev, 0)
                    tmp[pl.ds(0, 16)] = cs
                    cnt = tmp[pl.ds(0, 16)][15]
                    fill2 = fll + cnt

                    @pl.when(cnt > 0)
                    def _():
                        jv = jbuf[pl.ds(g * 16, 16)]
                        pp = jnp.zeros((16,), jnp.int32)
                        tgt = iot + 1
                        for sh in (8, 4, 2, 1):
                            probe = _vperm(cs, jnp.minimum(pp + sh - 1, 15))
                            pp = pp + jnp.where(probe < tgt, sh, 0)
                        pp = jnp.minimum(pp, 15)
                        eidv = k * SC_CH + g * 16 + iot
                        eid_q[pl.ds(fll, 16)] = _vperm(eidv, pp)
                        dst_q[pl.ds(fll, 16)] = _vperm(dstv, pp)
                        jq[pl.ds(fll, 16)] = _vperm(jv, pp)

                    @pl.when(fill2 >= C)
                    def _():
                        fire(C)
                        eid_q[pl.ds(0, 16)] = eid_q[pl.ds(C, 16)]
                        dst_q[pl.ds(0, 16)] = dst_q[pl.ds(C, 16)]
                        jq[pl.ds(0, 16)] = jq[pl.ds(C, 16)]

                    return jnp.where(fill2 >= C, fill2 - C, fill2)

                fill = lax.fori_loop(0, SC_CH // 16, grp, fill)
            return fill

        fill = lax.fori_loop(0, NCH // 2, step, jnp.int32(0))

        @pl.when(fill > 0)
        def _():
            fire(fill)

        pltpu.sync_copy(acc_dv, outv_hbm.at[pl.ds(lo, RN)])
        pltpu.sync_copy(acc_ds, outs_hbm.at[pl.ds(lo, RN)])
        return 0

    lax.fori_loop(0, 2, one_pass, 0)


@functools.partial(
    pl.kernel,
    out_type=(
        jax.ShapeDtypeStruct((NP, 3 * H), jnp.float32),
        jax.ShapeDtypeStruct((NP, H), jnp.float32),
    ),
    mesh=plsc.VectorSubcoreMesh(
        core_axis_name="c", subcore_axis_name="s", num_cores=2, num_subcores=16),
    scratch_types=[
        pltpu.VMEM((SC_CH,), jnp.int32),
        pltpu.VMEM((SC_CH,), jnp.int32),
        pltpu.VMEM((SC_CH,), jnp.int32),
        pltpu.VMEM((SC_CH,), jnp.int32),
        pltpu.VMEM((CAP,), jnp.int32),
        pltpu.VMEM((CAP,), jnp.int32),
        pltpu.VMEM((CAP,), jnp.int32),
        pltpu.VMEM((16,), jnp.int32),
        pltpu.VMEM((C,), jnp.int32),
        pltpu.VMEM((C,), jnp.int32),
        pltpu.VMEM((C, 3 * H), jnp.float32),
        pltpu.VMEM((C, PKW), jnp.float32),
        pltpu.VMEM((C, 3 * H), jnp.float32),
        pltpu.VMEM((RN, 3 * H), jnp.float32),
        pltpu.VMEM((RN, H), jnp.float32),
        pltpu.SemaphoreType.DMA,
        pltpu.SemaphoreType.DMA,
        pltpu.SemaphoreType.DMA,
        pltpu.SemaphoreType.DMA,
        pltpu.SemaphoreType.DMA,
        pltpu.SemaphoreType.DMA,
        pltpu.SemaphoreType.DMA,
    ],
)
def _sc_kernel(*args):
    _sc_body(*args)


def kernel(v, s, edge_index, phi_ij, d_ij, dir_ij, W1, b1, W2, b2, Wr, br):
    sp = jnp.pad(s, ((0, NP - N), (0, 0)))
    vp = jnp.pad(v.reshape(N, 3 * H), ((0, NP - N), (0, 0)))
    se = _dense(sp, W1, b1, W2, b2)
    phip = jnp.pad(phi_ij, ((0, E2 - E), (0, 0)))
    dp = jnp.pad(d_ij, ((0, E2 - E), (0, 0)), constant_values=2.0 * CUTOFF)
    dirp = jnp.pad(dir_ij, ((0, E2 - E), (0, 0)))
    pk = _pack(phip, dp, dirp, Wr, br)
    ip = jnp.pad(edge_index[0], (0, EP - E), constant_values=np.int32(2 ** 30))
    jp = jnp.pad(edge_index[1], (0, EP - E))
    outv, outs = _sc_kernel(se, pk, vp, sp, ip, jp)
    return (outv[:N].reshape(N, 3, H), outs[:N])


# speedup vs baseline: 1.1848x; 1.1848x over previous
import functools

import jax
import jax.numpy as jnp
import numpy as np
from jax import lax
from jax.experimental import pallas as pl
from jax.experimental.pallas import tpu as pltpu
from jax.experimental.pallas import tpu_sc as plsc

N = 10000
E = 160000
H = 128
RBF = 16
CUTOFF = 5.0

NP = 10240
PKW = 512
E2 = 162000
EP = 164864
SC_CH = 1792
NCH = EP // SC_CH
RN = 160
C = 32
CAP = 80


def _dense_body(s_ref, W1_ref, b1_ref, W2_ref, b2_ref, out_ref):
    h = s_ref[...] @ W1_ref[...] + b1_ref[...]
    h = h * jax.nn.sigmoid(h)
    out_ref[...] = h @ W2_ref[...] + b2_ref[...]


def _dense(s, W1, b1, W2, b2):
    BN = 1024
    return pl.pallas_call(
        _dense_body,
        grid=(NP // BN,),
        in_specs=[
            pl.BlockSpec((BN, H), lambda i: (i, 0)),
            pl.BlockSpec((H, H), lambda i: (0, 0)),
            pl.BlockSpec((1, H), lambda i: (0, 0)),
            pl.BlockSpec((H, 3 * H), lambda i: (0, 0)),
            pl.BlockSpec((1, 3 * H), lambda i: (0, 0)),
        ],
        out_specs=pl.BlockSpec((BN, 3 * H), lambda i: (i, 0)),
        out_shape=jax.ShapeDtypeStruct((NP, 3 * H), jnp.float32),
    )(s, W1, b1.reshape(1, H), W2, b2.reshape(1, 3 * H))


def _pack_body(phi_ref, d_ref, dir_ref, Wr_ref, br_ref, out_ref):
    r = phi_ref[...] @ Wr_ref[...] + br_ref[...]
    d = d_ref[...]
    cut = 0.5 * (jnp.cos(jnp.pi * d / CUTOFF) + 1.0) * (d < CUTOFF).astype(d.dtype)
    out_ref[:, : 3 * H] = r * cut
    be = r.shape[0]
    out_ref[:, 3 * H: 4 * H] = jnp.zeros((be, H), jnp.float32)
    for k in range(3):
        col = dir_ref[:, k:k + 1]
        out_ref[:, 3 * H + 16 * k: 3 * H + 16 * (k + 1)] = jnp.broadcast_to(
            col, (be, 16))


def _pack(phi, d, dir_ij, Wr, br):
    BE = 2000
    return pl.pallas_call(
        _pack_body,
        grid=(E2 // BE,),
        in_specs=[
            pl.BlockSpec((BE, RBF), lambda i: (i, 0)),
            pl.BlockSpec((BE, 1), lambda i: (i, 0)),
            pl.BlockSpec((BE, 3), lambda i: (i, 0)),
            pl.BlockSpec((RBF, 3 * H), lambda i: (0, 0)),
            pl.BlockSpec((1, 3 * H), lambda i: (0, 0)),
        ],
        out_specs=pl.BlockSpec((BE, PKW), lambda i: (i, 0)),
        out_shape=jax.ShapeDtypeStruct((E2, PKW), jnp.float32),
    )(phi, d, dir_ij, Wr, br.reshape(1, 3 * H))


_DN = lax.GatherDimensionNumbers(
    offset_dims=(), collapsed_slice_dims=(0,), start_index_map=(0,))


def _vperm(x, idx):
    return lax.gather(x, idx[:, None], _DN, (1,),
                      mode=lax.GatherScatterMode.PROMISE_IN_BOUNDS)


def _sc_body(se_hbm, pk_hbm, vp_hbm, sp_hbm, ip_hbm, jp_hbm,
             outv_hbm, outs_hbm,
             ib0, ib1, jb0, jb1, eid_q, dst_q, jq, tmp, ge, gj, gd,
             se_b, pk_b, v_b, acc_dv, acc_ds,
             semi0, semi1, semj0, semj1, sg1, sg2, sg3):
    c = lax.axis_index("c")
    s = lax.axis_index("s")
    w = c * 16 + s
    iot = lax.iota(jnp.int32, 16)

    ibufs = (ib0, ib1)
    jbufs = (jb0, jb1)
    semis = (semi0, semi1)
    semjs = (semj0, semj1)

    def fire_start(nv):
        for gi in range(C // 16):
            lanes = gi * 16 + iot
            val = lanes < nv
            e16 = jnp.where(val, eid_q[pl.ds(gi * 16, 16)], E)
            j16 = jnp.clip(jq[pl.ds(gi * 16, 16)], 0, NP - 1)
            d16 = jnp.clip(dst_q[pl.ds(gi * 16, 16)], 0, RN - 1)
            ge[pl.ds(gi * 16, 16)] = e16
            gj[pl.ds(gi * 16, 16)] = j16
            gd[pl.ds(gi * 16, 16)] = d16
        pltpu.async_copy(se_hbm.at[gj], se_b, sg1)
        pltpu.async_copy(pk_hbm.at[ge], pk_b, sg2)
        pltpu.async_copy(vp_hbm.at[gj], v_b, sg3)

    def fire_finish():
        pltpu.make_async_copy(se_hbm.at[gj], se_b, sg1).wait()
        pltpu.make_async_copy(pk_hbm.at[ge], pk_b, sg2).wait()
        pltpu.make_async_copy(vp_hbm.at[gj], v_b, sg3).wait()

        def sub(gi, _):
            d16 = gd[pl.ds(gi * 16, 16)]
            for kk in range(16):
                e = gi * 16 + kk
                dr = d16[kk]
                dirx = pk_b[e, pl.ds(3 * H, 16)]
                diry = pk_b[e, pl.ds(3 * H + 16, 16)]
                dirz = pk_b[e, pl.ds(3 * H + 32, 16)]

                def hcb(hc, _2, e=e, dr=dr, dirx=dirx, diry=diry, dirz=dirz):
                    o = hc * 16
                    g_dvv = se_b[e, pl.ds(o, 16)] * pk_b[e, pl.ds(o, 16)]
                    g_ds = se_b[e, pl.ds(H + o, 16)] * pk_b[e, pl.ds(H + o, 16)]
                    g_dvr = se_b[e, pl.ds(2 * H + o, 16)] * pk_b[e, pl.ds(2 * H + o, 16)]
                    plsc.addupdate(acc_ds.at[dr, pl.ds(o, 16)], g_ds)
                    plsc.addupdate(acc_dv.at[dr, pl.ds(o, 16)],
                                   g_dvr * dirx + g_dvv * v_b[e, pl.ds(o, 16)])
                    plsc.addupdate(acc_dv.at[dr, pl.ds(H + o, 16)],
                                   g_dvr * diry + g_dvv * v_b[e, pl.ds(H + o, 16)])
                    plsc.addupdate(acc_dv.at[dr, pl.ds(2 * H + o, 16)],
                                   g_dvr * dirz + g_dvv * v_b[e, pl.ds(2 * H + o, 16)])
                    return 0

                lax.fori_loop(0, H // 16, hcb, 0)
            return 0

        lax.fori_loop(0, C // 16, sub, 0)

    def prefix(m):
        cs = jnp.where(m, 1, 0)
        for sh in (1, 2, 4, 8):
            prev = _vperm(cs, jnp.maximum(iot - sh, 0))
            cs = cs + jnp.where(iot >= sh, prev, 0)
        return cs

    UNR = 2

    def one_pass(p, _):
        lo = (p * 32 + w) * RN
        pltpu.sync_copy(vp_hbm.at[pl.ds(lo, RN)], acc_dv)
        pltpu.sync_copy(sp_hbm.at[pl.ds(lo, RN)], acc_ds)
        pltpu.async_copy(ip_hbm.at[pl.ds(0, SC_CH)], ib0, semi0)
        pltpu.async_copy(jp_hbm.at[pl.ds(0, SC_CH)], jb0, semj0)

        def step(st, carry0):
            carry = carry0
            for par in range(2):
                k = st * 2 + par
                ibuf = ibufs[par]
                jbuf = jbufs[par]
                pltpu.make_async_copy(
                    ip_hbm.at[pl.ds(k * SC_CH, SC_CH)], ibuf, semis[par]).wait()
                pltpu.make_async_copy(
                    jp_hbm.at[pl.ds(k * SC_CH, SC_CH)], jbuf, semjs[par]).wait()

                @pl.when(k + 1 < NCH)
                def _():
                    nxt = (k + 1) * SC_CH
                    pltpu.async_copy(
                        ip_hbm.at[pl.ds(nxt, SC_CH)], ibufs[1 - par], semis[1 - par])
                    pltpu.async_copy(
                        jp_hbm.at[pl.ds(nxt, SC_CH)], jbufs[1 - par], semjs[1 - par])

                def grp(gg, cr):
                    fill, pending = cr
                    csl = []
                    dstl = []
                    for a in range(UNR):
                        iv = ibuf[pl.ds(gg * (UNR * 16) + a * 16, 16)]
                        dstv = iv - lo
                        m = (dstv >= 0) & (dstv < RN)
                        cs = prefix(m)
                        tmp[pl.ds(a * 16, 16)] = cs
                        csl.append(cs)
                        dstl.append(dstv)
                    fb = fill
                    for a in range(UNR):
                        cs = csl[a]
                        dstv = dstl[a]
                        cnt = tmp[pl.ds(a * 16, 16)][15]

                        @pl.when(cnt > 0)
                        def _(cs=cs, dstv=dstv, fb=fb, a=a):
                            jv = jbuf[pl.ds(gg * (UNR * 16) + a * 16, 16)]
                            pp = jnp.zeros((16,), jnp.int32)
                            tgt = iot + 1
                            for sh in (8, 4, 2, 1):
                                probe = _vperm(cs, jnp.minimum(pp + sh - 1, 15))
                                pp = pp + jnp.where(probe < tgt, sh, 0)
                            pp = jnp.minimum(pp, 15)
                            eidv = k * SC_CH + gg * (UNR * 16) + a * 16 + iot
                            eid_q[pl.ds(fb, 16)] = _vperm(eidv, pp)
                            dst_q[pl.ds(fb, 16)] = _vperm(dstv, pp)
                            jq[pl.ds(fb, 16)] = _vperm(jv, pp)

                        fb = fb + cnt
                    fill2 = fb

                    @pl.when(fill2 >= C)
                    def _(pending=pending):
                        @pl.when(pending == 1)
                        def _():
                            fire_finish()

                        fire_start(C)
                        eid_q[pl.ds(0, 16)] = eid_q[pl.ds(C, 16)]
                        dst_q[pl.ds(0, 16)] = dst_q[pl.ds(C, 16)]
                        jq[pl.ds(0, 16)] = jq[pl.ds(C, 16)]
                        eid_q[pl.ds(16, 16)] = eid_q[pl.ds(C + 16, 16)]
                        dst_q[pl.ds(16, 16)] = dst_q[pl.ds(C + 16, 16)]
                        jq[pl.ds(16, 16)] = jq[pl.ds(C + 16, 16)]

                    pending = jnp.where(fill2 >= C, jnp.int32(1), pending)
                    fill = jnp.where(fill2 >= C, fill2 - C, fill2)
                    return (fill, pending)

                carry = lax.fori_loop(0, SC_CH // (UNR * 16), grp, carry)
            return carry

        fill, pending = lax.fori_loop(
            0, NCH // 2, step, (jnp.int32(0), jnp.int32(0)))

        @pl.when(pending == 1)
        def _():
            fire_finish()

        @pl.when(fill > 0)
        def _():
            fire_start(fill)
            fire_finish()

        pltpu.sync_copy(acc_dv, outv_hbm.at[pl.ds(lo, RN)])
        pltpu.sync_copy(acc_ds, outs_hbm.at[pl.ds(lo, RN)])
        return 0

    lax.fori_loop(0, 2, one_pass, 0)


@functools.partial(
    pl.kernel,
    out_type=(
        jax.ShapeDtypeStruct((NP, 3 * H), jnp.float32),
        jax.ShapeDtypeStruct((NP, H), jnp.float32),
    ),
    mesh=plsc.VectorSubcoreMesh(
        core_axis_name="c", subcore_axis_name="s", num_cores=2, num_subcores=16),
    scratch_types=[
        pltpu.VMEM((SC_CH,), jnp.int32),
        pltpu.VMEM((SC_CH,), jnp.int32),
        pltpu.VMEM((SC_CH,), jnp.int32),
        pltpu.VMEM((SC_CH,), jnp.int32),
        pltpu.VMEM((CAP,), jnp.int32),
        pltpu.VMEM((CAP,), jnp.int32),
        pltpu.VMEM((CAP,), jnp.int32),
        pltpu.VMEM((64,), jnp.int32),
        pltpu.VMEM((C,), jnp.int32),
        pltpu.VMEM((C,), jnp.int32),
        pltpu.VMEM((C,), jnp.int32),
        pltpu.VMEM((C, 3 * H), jnp.float32),
        pltpu.VMEM((C, PKW), jnp.float32),
        pltpu.VMEM((C, 3 * H), jnp.float32),
        pltpu.VMEM((RN, 3 * H), jnp.float32),
        pltpu.VMEM((RN, H), jnp.float32),
        pltpu.SemaphoreType.DMA,
        pltpu.SemaphoreType.DMA,
        pltpu.SemaphoreType.DMA,
        pltpu.SemaphoreType.DMA,
        pltpu.SemaphoreType.DMA,
        pltpu.SemaphoreType.DMA,
        pltpu.SemaphoreType.DMA,
    ],
)
def _sc_kernel(*args):
    _sc_body(*args)


def kernel(v, s, edge_index, phi_ij, d_ij, dir_ij, W1, b1, W2, b2, Wr, br):
    sp = jnp.pad(s, ((0, NP - N), (0, 0)))
    vp = jnp.pad(v.reshape(N, 3 * H), ((0, NP - N), (0, 0)))
    se = _dense(sp, W1, b1, W2, b2)
    phip = jnp.pad(phi_ij, ((0, E2 - E), (0, 0)))
    dp = jnp.pad(d_ij, ((0, E2 - E), (0, 0)), constant_values=2.0 * CUTOFF)
    dirp = jnp.pad(dir_ij, ((0, E2 - E), (0, 0)))
    pk = _pack(phip, dp, dirp, Wr, br)
    ip = jnp.pad(edge_index[0], (0, EP - E), constant_values=np.int32(2 ** 30))
    jp = jnp.pad(edge_index[1], (0, EP - E))
    outv, outs = _sc_kernel(se, pk, vp, sp, ip, jp)
    return (outv[:N].reshape(N, 3, H), outs[:N])


# speedup vs baseline: 1.3277x; 1.1206x over previous
import functools

import jax
import jax.numpy as jnp
import numpy as np
from jax import lax
from jax.experimental import pallas as pl
from jax.experimental.pallas import tpu as pltpu
from jax.experimental.pallas import tpu_sc as plsc

N = 10000
E = 160000
H = 128
RBF = 16
CUTOFF = 5.0

NP = 10240
PKW = 512
E2 = 162000
EP = 164864
SC_CH = 1792
NCH = EP // SC_CH
RN = 160
C = 32
CAP = 80


def _dense_body(s_ref, W1_ref, b1_ref, W2_ref, b2_ref, out_ref):
    h = s_ref[...] @ W1_ref[...] + b1_ref[...]
    h = h * jax.nn.sigmoid(h)
    out_ref[...] = h @ W2_ref[...] + b2_ref[...]


def _dense(s, W1, b1, W2, b2):
    BN = 1024
    return pl.pallas_call(
        _dense_body,
        grid=(NP // BN,),
        in_specs=[
            pl.BlockSpec((BN, H), lambda i: (i, 0)),
            pl.BlockSpec((H, H), lambda i: (0, 0)),
            pl.BlockSpec((1, H), lambda i: (0, 0)),
            pl.BlockSpec((H, 3 * H), lambda i: (0, 0)),
            pl.BlockSpec((1, 3 * H), lambda i: (0, 0)),
        ],
        out_specs=pl.BlockSpec((BN, 3 * H), lambda i: (i, 0)),
        out_shape=jax.ShapeDtypeStruct((NP, 3 * H), jnp.float32),
    )(s, W1, b1.reshape(1, H), W2, b2.reshape(1, 3 * H))


def _pack_body(phic_ref, dir_ref, Wr2_ref, out_ref):
    out_ref[:, : 3 * H] = phic_ref[...] @ Wr2_ref[...]
    out_ref[:, 3 * H:] = dir_ref[...]


def _pack(phic, dir_rep, Wr2):
    BE = 2000
    return pl.pallas_call(
        _pack_body,
        grid=(E2 // BE,),
        in_specs=[
            pl.BlockSpec((BE, 2 * RBF), lambda i: (i, 0)),
            pl.BlockSpec((BE, H), lambda i: (i, 0)),
            pl.BlockSpec((2 * RBF, 3 * H), lambda i: (0, 0)),
        ],
        out_specs=pl.BlockSpec((BE, PKW), lambda i: (i, 0)),
        out_shape=jax.ShapeDtypeStruct((E2, PKW), jnp.float32),
    )(phic, dir_rep, Wr2)


_DN = lax.GatherDimensionNumbers(
    offset_dims=(), collapsed_slice_dims=(0,), start_index_map=(0,))


def _vperm(x, idx):
    return lax.gather(x, idx[:, None], _DN, (1,),
                      mode=lax.GatherScatterMode.PROMISE_IN_BOUNDS)


def _sc_body(se_hbm, pk_hbm, vp_hbm, sp_hbm, ip_hbm, jp_hbm,
             outv_hbm, outs_hbm,
             ib0, ib1, jb0, jb1, eid_q, dst_q, jq, tmp, ge, gj, gd,
             se_b, pk_b, v_b, acc_dv, acc_ds,
             semi0, semi1, semj0, semj1, sg1, sg2, sg3):
    c = lax.axis_index("c")
    s = lax.axis_index("s")
    w = c * 16 + s
    iot = lax.iota(jnp.int32, 16)

    ibufs = (ib0, ib1)
    jbufs = (jb0, jb1)
    semis = (semi0, semi1)
    semjs = (semj0, semj1)

    def fire_start(nv):
        for gi in range(C // 16):
            lanes = gi * 16 + iot
            val = lanes < nv
            e16 = jnp.where(val, eid_q[pl.ds(gi * 16, 16)], E)
            j16 = jnp.clip(jq[pl.ds(gi * 16, 16)], 0, NP - 1)
            d16 = jnp.clip(dst_q[pl.ds(gi * 16, 16)], 0, RN - 1)
            ge[pl.ds(gi * 16, 16)] = e16
            gj[pl.ds(gi * 16, 16)] = j16
            gd[pl.ds(gi * 16, 16)] = d16
        pltpu.async_copy(se_hbm.at[gj], se_b, sg1)
        pltpu.async_copy(pk_hbm.at[ge], pk_b, sg2)
        pltpu.async_copy(vp_hbm.at[gj], v_b, sg3)

    def fire_finish():
        pltpu.make_async_copy(se_hbm.at[gj], se_b, sg1).wait()
        pltpu.make_async_copy(pk_hbm.at[ge], pk_b, sg2).wait()
        pltpu.make_async_copy(vp_hbm.at[gj], v_b, sg3).wait()

        def sub(gi, _):
            d16 = gd[pl.ds(gi * 16, 16)]
            for kk in range(16):
                e = gi * 16 + kk
                dr = d16[kk]
                dirx = pk_b[e, pl.ds(3 * H, 16)]
                diry = pk_b[e, pl.ds(3 * H + 16, 16)]
                dirz = pk_b[e, pl.ds(3 * H + 32, 16)]

                def hcb(hc, _2, e=e, dr=dr, dirx=dirx, diry=diry, dirz=dirz):
                    o = hc * 16
                    g_dvv = se_b[e, pl.ds(o, 16)] * pk_b[e, pl.ds(o, 16)]
                    g_ds = se_b[e, pl.ds(H + o, 16)] * pk_b[e, pl.ds(H + o, 16)]
                    g_dvr = se_b[e, pl.ds(2 * H + o, 16)] * pk_b[e, pl.ds(2 * H + o, 16)]
                    plsc.addupdate(acc_ds.at[dr, pl.ds(o, 16)], g_ds)
                    plsc.addupdate(acc_dv.at[dr, pl.ds(o, 16)],
                                   g_dvr * dirx + g_dvv * v_b[e, pl.ds(o, 16)])
                    plsc.addupdate(acc_dv.at[dr, pl.ds(H + o, 16)],
                                   g_dvr * diry + g_dvv * v_b[e, pl.ds(H + o, 16)])
                    plsc.addupdate(acc_dv.at[dr, pl.ds(2 * H + o, 16)],
                                   g_dvr * dirz + g_dvv * v_b[e, pl.ds(2 * H + o, 16)])
                    return 0

                lax.fori_loop(0, H // 16, hcb, 0)
            return 0

        lax.fori_loop(0, C // 16, sub, 0)

    def prefix(m):
        cs = jnp.where(m, 1, 0)
        for sh in (1, 2, 4, 8):
            prev = _vperm(cs, jnp.maximum(iot - sh, 0))
            cs = cs + jnp.where(iot >= sh, prev, 0)
        return cs

    UNR = 2

    def one_pass(p, _):
        lo = (p * 32 + w) * RN
        pltpu.sync_copy(vp_hbm.at[pl.ds(lo, RN)], acc_dv)
        pltpu.sync_copy(sp_hbm.at[pl.ds(lo, RN)], acc_ds)
        pltpu.async_copy(ip_hbm.at[pl.ds(0, SC_CH)], ib0, semi0)
        pltpu.async_copy(jp_hbm.at[pl.ds(0, SC_CH)], jb0, semj0)

        def step(st, carry0):
            carry = carry0
            for par in range(2):
                k = st * 2 + par
                ibuf = ibufs[par]
                jbuf = jbufs[par]
                pltpu.make_async_copy(
                    ip_hbm.at[pl.ds(k * SC_CH, SC_CH)], ibuf, semis[par]).wait()
                pltpu.make_async_copy(
                    jp_hbm.at[pl.ds(k * SC_CH, SC_CH)], jbuf, semjs[par]).wait()

                @pl.when(k + 1 < NCH)
                def _():
                    nxt = (k + 1) * SC_CH
                    pltpu.async_copy(
                        ip_hbm.at[pl.ds(nxt, SC_CH)], ibufs[1 - par], semis[1 - par])
                    pltpu.async_copy(
                        jp_hbm.at[pl.ds(nxt, SC_CH)], jbufs[1 - par], semjs[1 - par])

                def grp(gg, cr):
                    fill, pending = cr
                    csl = []
                    dstl = []
                    for a in range(UNR):
                        iv = ibuf[pl.ds(gg * (UNR * 16) + a * 16, 16)]
                        dstv = iv - lo
                        m = (dstv >= 0) & (dstv < RN)
                        cs = prefix(m)
                        tmp[pl.ds(a * 16, 16)] = cs
                        csl.append(cs)
                        dstl.append(dstv)
                    fb = fill
                    for a in range(UNR):
                        cs = csl[a]
                        dstv = dstl[a]
                        cnt = tmp[pl.ds(a * 16, 16)][15]

                        @pl.when(cnt > 0)
                        def _(cs=cs, dstv=dstv, fb=fb, a=a):
                            jv = jbuf[pl.ds(gg * (UNR * 16) + a * 16, 16)]
                            pp = jnp.zeros((16,), jnp.int32)
                            tgt = iot + 1
                            for sh in (8, 4, 2, 1):
                                probe = _vperm(cs, jnp.minimum(pp + sh - 1, 15))
                                pp = pp + jnp.where(probe < tgt, sh, 0)
                            pp = jnp.minimum(pp, 15)
                            eidv = k * SC_CH + gg * (UNR * 16) + a * 16 + iot
                            eid_q[pl.ds(fb, 16)] = _vperm(eidv, pp)
                            dst_q[pl.ds(fb, 16)] = _vperm(dstv, pp)
                            jq[pl.ds(fb, 16)] = _vperm(jv, pp)

                        fb = fb + cnt
                    fill2 = fb

                    @pl.when(fill2 >= C)
                    def _(pending=pending):
                        @pl.when(pending == 1)
                        def _():
                            fire_finish()

                        fire_start(C)
                        eid_q[pl.ds(0, 16)] = eid_q[pl.ds(C, 16)]
                        dst_q[pl.ds(0, 16)] = dst_q[pl.ds(C, 16)]
                        jq[pl.ds(0, 16)] = jq[pl.ds(C, 16)]
                        eid_q[pl.ds(16, 16)] = eid_q[pl.ds(C + 16, 16)]
                        dst_q[pl.ds(16, 16)] = dst_q[pl.ds(C + 16, 16)]
                        jq[pl.ds(16, 16)] = jq[pl.ds(C + 16, 16)]

                    pending = jnp.where(fill2 >= C, jnp.int32(1), pending)
                    fill = jnp.where(fill2 >= C, fill2 - C, fill2)
                    return (fill, pending)

                carry = lax.fori_loop(0, SC_CH // (UNR * 16), grp, carry)
            return carry

        fill, pending = lax.fori_loop(
            0, NCH // 2, step, (jnp.int32(0), jnp.int32(0)))

        @pl.when(pending == 1)
        def _():
            fire_finish()

        @pl.when(fill > 0)
        def _():
            fire_start(fill)
            fire_finish()

        pltpu.sync_copy(acc_dv, outv_hbm.at[pl.ds(lo, RN)])
        pltpu.sync_copy(acc_ds, outs_hbm.at[pl.ds(lo, RN)])
        return 0

    lax.fori_loop(0, 2, one_pass, 0)


@functools.partial(
    pl.kernel,
    out_type=(
        jax.ShapeDtypeStruct((NP, 3 * H), jnp.float32),
        jax.ShapeDtypeStruct((NP, H), jnp.float32),
    ),
    mesh=plsc.VectorSubcoreMesh(
        core_axis_name="c", subcore_axis_name="s", num_cores=2, num_subcores=16),
    scratch_types=[
        pltpu.VMEM((SC_CH,), jnp.int32),
        pltpu.VMEM((SC_CH,), jnp.int32),
        pltpu.VMEM((SC_CH,), jnp.int32),
        pltpu.VMEM((SC_CH,), jnp.int32),
        pltpu.VMEM((CAP,), jnp.int32),
        pltpu.VMEM((CAP,), jnp.int32),
        pltpu.VMEM((CAP,), jnp.int32),
        pltpu.VMEM((64,), jnp.int32),
        pltpu.VMEM((C,), jnp.int32),
        pltpu.VMEM((C,), jnp.int32),
        pltpu.VMEM((C,), jnp.int32),
        pltpu.VMEM((C, 3 * H), jnp.float32),
        pltpu.VMEM((C, PKW), jnp.float32),
        pltpu.VMEM((C, 3 * H), jnp.float32),
        pltpu.VMEM((RN, 3 * H), jnp.float32),
        pltpu.VMEM((RN, H), jnp.float32),
        pltpu.SemaphoreType.DMA,
        pltpu.SemaphoreType.DMA,
        pltpu.SemaphoreType.DMA,
        pltpu.SemaphoreType.DMA,
        pltpu.SemaphoreType.DMA,
        pltpu.SemaphoreType.DMA,
        pltpu.SemaphoreType.DMA,
    ],
)
def _sc_kernel(*args):
    _sc_body(*args)


def kernel(v, s, edge_index, phi_ij, d_ij, dir_ij, W1, b1, W2, b2, Wr, br):
    sp = jnp.pad(s, ((0, NP - N), (0, 0)))
    vp = jnp.pad(v.reshape(N, 3 * H), ((0, NP - N), (0, 0)))
    se = _dense(sp, W1, b1, W2, b2)
    phip = jnp.pad(phi_ij, ((0, E2 - E), (0, 0)))
    dp = jnp.pad(d_ij, ((0, E2 - E), (0, 0)), constant_values=2.0 * CUTOFF)
    cut = 0.5 * (jnp.cos(jnp.pi * dp / CUTOFF) + 1.0) * (dp < CUTOFF)
    phic = jnp.concatenate(
        [phip * cut, cut, jnp.zeros((E2, RBF - 1), jnp.float32)], axis=1)
    Wr2 = jnp.concatenate(
        [Wr, br[None, :], jnp.zeros((RBF - 1, 3 * H), jnp.float32)], axis=0)
    dir_rep = jnp.pad(jnp.repeat(dir_ij, 16, axis=1),
                      ((0, E2 - E), (0, H - 48)))
    pk = _pack(phic, dir_rep, Wr2)
    ip = jnp.pad(edge_index[0], (0, EP - E), constant_values=np.int32(2 ** 30))
    jp = jnp.pad(edge_index[1], (0, EP - E))
    outv, outs = _sc_kernel(se, pk, vp, sp, ip, jp)
    return (outv[:N].reshape(N, 3, H), outs[:N])
